# speedup vs baseline: 1.0722x; 1.0722x over previous
import functools

import jax
import jax.numpy as jnp
from jax import lax
from jax.experimental import pallas as pl
from jax.experimental.pallas import tpu as pltpu
from jax.experimental.pallas import tpu_sc as plsc

HIDDEN = 1024
EPS = 1e-12

NC = 2
NS = 16
NW = NC * NS

CHUNK = 64


def _gather_sc(table, idx):
    b = idx.shape[0]
    b_per_w = b // NW
    n_chunks = b_per_w // CHUNK
    mesh = plsc.VectorSubcoreMesh(core_axis_name="c", subcore_axis_name="s")

    @functools.partial(
        pl.kernel,
        mesh=mesh,
        out_type=jax.ShapeDtypeStruct((b, HIDDEN), jnp.float32),
        scratch_types=[
            pltpu.VMEM((b_per_w,), jnp.int32),
            pltpu.VMEM((CHUNK, HIDDEN), jnp.float32),
            pltpu.SemaphoreType.DMA,
        ],
    )
    def k(table_hbm, idx_hbm, out_hbm, idx_v, rows_v, sem):
        wid = lax.axis_index("s") * NC + lax.axis_index("c")
        base = wid * b_per_w
        pltpu.sync_copy(idx_hbm.at[pl.ds(base, b_per_w)], idx_v)

        @pl.loop(0, n_chunks)
        def _(i):
            off = i * CHUNK
            pltpu.async_copy(
                table_hbm.at[idx_v.at[pl.ds(off, CHUNK)]], rows_v, sem
            ).wait()
            pltpu.sync_copy(rows_v, out_hbm.at[pl.ds(base + off, CHUNK)])

    return k(table, idx)


def _layernorm_tc(x, gamma, beta):
    b = x.shape[0]
    bt = 256

    def body(x_ref, g_ref, b_ref, o_ref):
        v = x_ref[...]
        m = jnp.mean(v, axis=1, keepdims=True)
        c = v - m
        var = jnp.mean(c * c, axis=1, keepdims=True)
        o_ref[...] = c * lax.rsqrt(var + EPS) * g_ref[...] + b_ref[...]

    return pl.pallas_call(
        body,
        grid=(b // bt,),
        in_specs=[
            pl.BlockSpec((bt, HIDDEN), lambda i: (i, 0)),
            pl.BlockSpec((1, HIDDEN), lambda i: (0, 0)),
            pl.BlockSpec((1, HIDDEN), lambda i: (0, 0)),
        ],
        out_specs=pl.BlockSpec((bt, HIDDEN), lambda i: (i, 0)),
        out_shape=jax.ShapeDtypeStruct((b, HIDDEN), jnp.float32),
    )(x, gamma.reshape(1, HIDDEN), beta.reshape(1, HIDDEN))


def kernel(input_ids, table, gamma, beta):
    bsh = input_ids.shape
    idx = input_ids.reshape(-1).astype(jnp.int32)
    gathered = _gather_sc(table, idx)
    out = _layernorm_tc(gathered, gamma, beta)
    return out.reshape(*bsh, HIDDEN)
